# baseline (speedup 1.0000x reference)
import jax
import jax.numpy as jnp
from jax import lax
from jax.experimental import pallas as pl
from jax.experimental.pallas import tpu as pltpu
from jax.experimental.pallas import tpu_sc as plsc

N = 10000
D = 128
H = 4
Q = 8
QH = Q * H
OUT = 256
HALF = OUT // 2
E = 320000

NC = 2
NS = 16
EPT = E // NS
B = 80
NCHUNK = EPT // B

RPT = 624
RW = 48
NWB = RPT // RW
TAIL0 = NS * RPT
TAILN = N - TAIL0

BLK = 1000
ACOLS = 8
WCOLS = OUT + 8


def _tc_body(x_ref, w_ref, b_ref, h_ref, an_ref):
    y = jnp.dot(x_ref[:], w_ref[:].T, preferred_element_type=jnp.float32)
    y = y + b_ref[:]
    h_ref[0] = y[:, :HALF].astype(jnp.bfloat16)
    h_ref[1] = y[:, HALF:OUT].astype(jnp.bfloat16)
    z = jnp.zeros((BLK, ACOLS - 4), jnp.float32)
    an_ref[0] = jnp.concatenate([y[:, OUT:OUT + 4], z], axis=1)
    an_ref[1] = jnp.concatenate([y[:, OUT + 4:OUT + 8], z], axis=1)


_tc_call = pl.pallas_call(
    _tc_body,
    grid=(N // BLK,),
    in_specs=[
        pl.BlockSpec((BLK, D), lambda j: (j, 0)),
        pl.BlockSpec((WCOLS, D), lambda j: (0, 0)),
        pl.BlockSpec((1, WCOLS), lambda j: (0, 0)),
    ],
    out_specs=[
        pl.BlockSpec((2, BLK, HALF), lambda j: (0, j, 0)),
        pl.BlockSpec((2, BLK, ACOLS), lambda j: (0, j, 0)),
    ],
    out_shape=[
        jax.ShapeDtypeStruct((2, N, HALF), jnp.bfloat16),
        jax.ShapeDtypeStruct((2, N, ACOLS), jnp.float32),
    ],
)


def _sc_body(hperm, anode, s_hbm, r_hbm, out_hbm,
             bufs, sbufs, wb_v, agg_sh, sems, ssems):
    c = lax.axis_index("c")
    sid = lax.axis_index("s")
    cN = c * N

    def _zero_row(i, carry):
        for q in range(8):
            wb_v[i, pl.ds(q * 16, 16)] = jnp.zeros((16,), jnp.float32)
        return carry
    lax.fori_loop(0, RW, _zero_row, 0)
    for t in range(NWB):
        pltpu.sync_copy(wb_v, agg_sh.at[pl.ds(sid * RPT + t * RW, RW)])
    @pl.when(sid == 0)
    def _zero_tail():
        pltpu.sync_copy(wb_v.at[pl.ds(0, TAILN)], agg_sh.at[pl.ds(TAIL0, TAILN)])
    plsc.subcore_barrier()

    def _fire_idx(k, buf, ss):
        sidx_v, ridx_v = buf[0], buf[1]
        base = sid * EPT + k * B
        pltpu.async_copy(s_hbm.at[pl.ds(base, B)], sidx_v, ss[6])
        pltpu.async_copy(r_hbm.at[pl.ds(base, B)], ridx_v, ss[7])

    def _fire_gathers(k, buf, ss):
        (sidx_v, ridx_v, i0_v, i1_v, i2_v, i3_v,
         as0_v, as1_v, ar0_v, ar1_v, a0b_v, a1b_v, rows_v) = buf
        base = sid * EPT + k * B
        pltpu.make_async_copy(s_hbm.at[pl.ds(base, B)], sidx_v, ss[6]).wait()
        pltpu.make_async_copy(r_hbm.at[pl.ds(base, B)], ridx_v, ss[7]).wait()
        for t in range(B // 16):
            sl = pl.ds(t * 16, 16)
            sa = sidx_v[sl] + cN
            sidx_v[sl] = sa
            s8 = sa * ACOLS
            i0_v[sl] = s8
            i1_v[sl] = s8 + 1
            r8 = (ridx_v[sl] + cN) * ACOLS
            i2_v[sl] = r8 + 2
            i3_v[sl] = r8 + 3
        pltpu.async_copy(hperm.at[sidx_v], rows_v, ss[0])
        pltpu.async_copy(anode.at[i0_v], as0_v, ss[1])
        pltpu.async_copy(anode.at[i1_v], as1_v, ss[2])
        pltpu.async_copy(anode.at[i2_v], ar0_v, ss[3])
        pltpu.async_copy(anode.at[i3_v], ar1_v, ss[4])

    def _wait_scatter(sbuf, ssem):
        pltpu.make_async_copy(sbuf[0], agg_sh.at[sbuf[1]], ssem).wait()

    def _process(k, buf, ss, sbuf, ssem):
        (sidx_v, ridx_v, i0_v, i1_v, i2_v, i3_v,
         as0_v, as1_v, ar0_v, ar1_v, a0b_v, a1b_v, rows_v) = buf
        msg_v, ridxsc_v = sbuf
        pltpu.make_async_copy(anode.at[i0_v], as0_v, ss[1]).wait()
        pltpu.make_async_copy(anode.at[i1_v], as1_v, ss[2]).wait()
        pltpu.make_async_copy(anode.at[i2_v], ar0_v, ss[3]).wait()
        pltpu.make_async_copy(anode.at[i3_v], ar1_v, ss[4]).wait()
        for t in range(B // 16):
            sl = pl.ds(t * 16, 16)
            a0b_v[sl] = as0_v[sl] + ar0_v[sl]
            a1b_v[sl] = as1_v[sl] + ar1_v[sl]
        @pl.when(k >= 2)
        def _drain():
            _wait_scatter(sbuf, ssem)
        for t in range(B // 16):
            sl = pl.ds(t * 16, 16)
            ridxsc_v[sl] = ridx_v[sl]
        pltpu.make_async_copy(hperm.at[sidx_v], rows_v, ss[0]).wait()

        def _edge(e, ecarry):
            va0 = jnp.full((16,), a0b_v[pl.ds(e, 16)][0], jnp.float32)
            va1 = jnp.full((16,), a1b_v[pl.ds(e, 16)][0], jnp.float32)
            for q in range(4):
                v32 = plsc.bitcast(rows_v[e, pl.ds(q * 16, 16)], jnp.bfloat16)
                lo, hi = plsc.unpack(v32, format=plsc.PackFormat.INTERLEAVED,
                                     preferred_element_type=jnp.float32)
                va = va0 if q < 2 else va1
                msg_v[e, pl.ds(q * 32, 16)] = lo * va
                msg_v[e, pl.ds(q * 32 + 16, 16)] = hi * va
            return ecarry
        lax.fori_loop(0, B, _edge, 0, unroll=8)
        pltpu.make_async_copy(msg_v, agg_sh.at[ridxsc_v], ssem).start(add=True)

    _fire_idx(0, bufs[0], sems[0])
    _fire_idx(1, bufs[1], sems[1])
    _fire_gathers(0, bufs[0], sems[0])

    def _group(g, carry):
        for b in range(6):
            k = g * 6 + b
            nb = (b + 1) % 3
            nb2 = (b + 2) % 3
            _fire_gathers(k + 1, bufs[nb], sems[nb])
            @pl.when(k < NCHUNK - 2)
            def _prefetch():
                _fire_idx(k + 2, bufs[nb2], sems[nb2])
            _process(k, bufs[b % 3], sems[b % 3], sbufs[b % 2], ssems[b % 2])
        return carry
    lax.fori_loop(0, (NCHUNK - 4) // 6, _group, 0)
    for k in range(NCHUNK - 4, NCHUNK):
        b = k % 6
        nb = (b + 1) % 3
        nb2 = (b + 2) % 3
        if k + 1 < NCHUNK:
            _fire_gathers(k + 1, bufs[nb], sems[nb])
        if k + 2 < NCHUNK:
            _fire_idx(k + 2, bufs[nb2], sems[nb2])
        _process(k, bufs[b % 3], sems[b % 3], sbufs[b % 2], ssems[b % 2])
    for p in range(2):
        _wait_scatter(sbufs[p], ssems[p])
    plsc.subcore_barrier()

    def _silu_rows(nrows):
        def _silu_row(i, carry):
            for q in range(8):
                sl = pl.ds(q * 16, 16)
                v = wb_v[i, sl]
                wb_v[i, sl] = v / (1.0 + jnp.exp(-v))
            return carry
        lax.fori_loop(0, nrows, _silu_row, 0)

    for t in range(NWB):
        r0 = sid * RPT + t * RW
        pltpu.sync_copy(agg_sh.at[pl.ds(r0, RW)], wb_v)
        _silu_rows(RW)
        pltpu.sync_copy(wb_v, out_hbm.at[pl.ds(r0, RW), pl.ds(c * HALF, HALF)])

    @pl.when(sid == 0)
    def _wb_tail():
        pltpu.sync_copy(agg_sh.at[pl.ds(TAIL0, TAILN)], wb_v.at[pl.ds(0, TAILN)])
        _silu_rows(TAILN)
        pltpu.sync_copy(wb_v.at[pl.ds(0, TAILN)],
                        out_hbm.at[pl.ds(TAIL0, TAILN), pl.ds(c * HALF, HALF)])


_sc_call = pl.kernel(
    _sc_body,
    out_type=jax.ShapeDtypeStruct((N, OUT), jnp.float32),
    mesh=plsc.VectorSubcoreMesh(
        core_axis_name="c", subcore_axis_name="s",
        num_cores=NC, num_subcores=NS),
    compiler_params=pltpu.CompilerParams(needs_layout_passes=False,
                                         use_tc_tiling_on_sc=False),
    scratch_types=[
        [
            [
                pltpu.VMEM((B,), jnp.int32),
                pltpu.VMEM((B,), jnp.int32),
                pltpu.VMEM((B,), jnp.int32),
                pltpu.VMEM((B,), jnp.int32),
                pltpu.VMEM((B,), jnp.int32),
                pltpu.VMEM((B,), jnp.int32),
                pltpu.VMEM((B,), jnp.float32),
                pltpu.VMEM((B,), jnp.float32),
                pltpu.VMEM((B,), jnp.float32),
                pltpu.VMEM((B,), jnp.float32),
                pltpu.VMEM((B + 16,), jnp.float32),
                pltpu.VMEM((B + 16,), jnp.float32),
                pltpu.VMEM((B, HALF // 2), jnp.float32),
            ]
            for _ in range(3)
        ],
        [
            [
                pltpu.VMEM((B, HALF), jnp.float32),
                pltpu.VMEM((B,), jnp.int32),
            ]
            for _ in range(2)
        ],
        pltpu.VMEM((RW, HALF), jnp.float32),
        pltpu.VMEM_SHARED((N, HALF), jnp.float32),
        [[pltpu.SemaphoreType.DMA] * 8 for _ in range(3)],
        [pltpu.SemaphoreType.DMA for _ in range(2)],
    ],
)


def _h_perm():
    perm = []
    for cc in (0, 1):
        for qq in range(4):
            base = cc * HALF + 32 * qq
            for m in range(16):
                perm += [base + m, base + 16 + m]
    return perm


def kernel(x, edge_index, W_w, W_b, a_w, a_b, lin_w, lin_b):
    x = x.astype(jnp.float32)
    s32 = edge_index[0].astype(jnp.int32)
    r32 = edge_index[1].astype(jnp.int32)

    a_ws, a_wr = a_w[:, :QH], a_w[:, QH:]
    mats, biases = [], []
    for cc in (0, 1):
        P = jnp.concatenate([a_ws[2 * cc:2 * cc + 2], a_wr[2 * cc:2 * cc + 2]], 0)
        mats.append(P @ W_w)
        bias = P @ W_b
        bias = bias.at[2].add(a_b[2 * cc]).at[3].add(a_b[2 * cc + 1])
        biases.append(bias)
    perm = jnp.array(_h_perm(), dtype=jnp.int32)
    WBIG = jnp.concatenate([lin_w[perm]] + mats, 0)
    bbig = jnp.concatenate([lin_b[perm]] + biases, 0)[None, :]

    h_perm, anode = _tc_call(x, WBIG, bbig)
    h32 = lax.bitcast_convert_type(
        h_perm.reshape(2 * N, HALF // 2, 2), jnp.float32)
    return _sc_call(h32, anode.reshape(2 * N * ACOLS), s32, r32)


# speedup vs baseline: 1.9362x; 1.9362x over previous
import jax
import jax.numpy as jnp
from jax import lax
from jax.experimental import pallas as pl
from jax.experimental.pallas import tpu as pltpu
from jax.experimental.pallas import tpu_sc as plsc

N = 10000
D = 128
H = 4
Q = 8
QH = Q * H
OUT = 256
HALF = OUT // 2
E = 320000

NC = 2
NS = 16
EPT = E // NS
B = 80
NCHUNK = EPT // B

RPT = 624
RW = 48
NWB = RPT // RW
TAIL0 = NS * RPT
TAILN = N - TAIL0

BLK = 1000
ACOLS = 8
WCOLS = OUT + 8


def _tc_body(x_ref, w_ref, b_ref, h_ref, an_ref):
    y = jnp.dot(x_ref[:], w_ref[:].T, preferred_element_type=jnp.float32)
    y = y + b_ref[:]
    h_ref[0] = y[:, :HALF]
    h_ref[1] = y[:, HALF:OUT]
    z = jnp.zeros((BLK, ACOLS - 4), jnp.float32)
    an_ref[0] = jnp.concatenate([y[:, OUT:OUT + 4], z], axis=1)
    an_ref[1] = jnp.concatenate([y[:, OUT + 4:OUT + 8], z], axis=1)


_tc_call = pl.pallas_call(
    _tc_body,
    grid=(N // BLK,),
    in_specs=[
        pl.BlockSpec((BLK, D), lambda j: (j, 0)),
        pl.BlockSpec((WCOLS, D), lambda j: (0, 0)),
        pl.BlockSpec((1, WCOLS), lambda j: (0, 0)),
    ],
    out_specs=[
        pl.BlockSpec((2, BLK, HALF), lambda j: (0, j, 0)),
        pl.BlockSpec((2, BLK, ACOLS), lambda j: (0, j, 0)),
    ],
    out_shape=[
        jax.ShapeDtypeStruct((2, N, HALF), jnp.float32),
        jax.ShapeDtypeStruct((2, N, ACOLS), jnp.float32),
    ],
)


def _sc_body(hperm, anode, s_hbm, r_hbm, out_hbm,
             bufs, wb_v, agg_sh, sems):
    c = lax.axis_index("c")
    sid = lax.axis_index("s")
    cN = c * N

    def _zero_row(i, carry):
        for q in range(8):
            wb_v[i, pl.ds(q * 16, 16)] = jnp.zeros((16,), jnp.float32)
        return carry
    lax.fori_loop(0, RW, _zero_row, 0)
    for t in range(NWB):
        pltpu.sync_copy(wb_v, agg_sh.at[pl.ds(sid * RPT + t * RW, RW)])
    @pl.when(sid == 0)
    def _zero_tail():
        pltpu.sync_copy(wb_v.at[pl.ds(0, TAILN)], agg_sh.at[pl.ds(TAIL0, TAILN)])
    plsc.subcore_barrier()

    def _fire_idx(k, buf, ss):
        sidx_v, ridx_v = buf[0], buf[1]
        base = sid * EPT + k * B
        pltpu.async_copy(s_hbm.at[pl.ds(base, B)], sidx_v, ss[6])
        pltpu.async_copy(r_hbm.at[pl.ds(base, B)], ridx_v, ss[7])

    def _fire_gathers(k, buf, ss):
        (sidx_v, ridx_v, i0_v, i1_v, i2_v, i3_v,
         as0_v, as1_v, ar0_v, ar1_v, a0b_v, a1b_v, rows_v, ridxsc_v) = buf
        base = sid * EPT + k * B
        pltpu.make_async_copy(s_hbm.at[pl.ds(base, B)], sidx_v, ss[6]).wait()
        pltpu.make_async_copy(r_hbm.at[pl.ds(base, B)], ridx_v, ss[7]).wait()
        for t in range(B // 16):
            sl = pl.ds(t * 16, 16)
            sa = sidx_v[sl] + cN
            sidx_v[sl] = sa
            s8 = sa * ACOLS
            i0_v[sl] = s8
            i1_v[sl] = s8 + 1
            rv = ridx_v[sl]
            ridxsc_v[sl] = rv
            r8 = (rv + cN) * ACOLS
            i2_v[sl] = r8 + 2
            i3_v[sl] = r8 + 3
        pltpu.async_copy(hperm.at[sidx_v], rows_v, ss[0])
        pltpu.async_copy(anode.at[i0_v], as0_v, ss[1])
        pltpu.async_copy(anode.at[i1_v], as1_v, ss[2])
        pltpu.async_copy(anode.at[i2_v], ar0_v, ss[3])
        pltpu.async_copy(anode.at[i3_v], ar1_v, ss[4])

    def _process(buf, ss):
        (sidx_v, ridx_v, i0_v, i1_v, i2_v, i3_v,
         as0_v, as1_v, ar0_v, ar1_v, a0b_v, a1b_v, rows_v, ridxsc_v) = buf
        pltpu.make_async_copy(anode.at[i0_v], as0_v, ss[1]).wait()
        pltpu.make_async_copy(anode.at[i1_v], as1_v, ss[2]).wait()
        pltpu.make_async_copy(anode.at[i2_v], ar0_v, ss[3]).wait()
        pltpu.make_async_copy(anode.at[i3_v], ar1_v, ss[4]).wait()
        for t in range(B // 16):
            sl = pl.ds(t * 16, 16)
            a0b_v[sl] = as0_v[sl] + ar0_v[sl]
            a1b_v[sl] = as1_v[sl] + ar1_v[sl]
        pltpu.make_async_copy(hperm.at[sidx_v], rows_v, ss[0]).wait()

        def _edge(e, ecarry):
            va0 = jnp.full((16,), a0b_v[pl.ds(e, 16)][0], jnp.float32)
            va1 = jnp.full((16,), a1b_v[pl.ds(e, 16)][0], jnp.float32)
            for q in range(4):
                sl = pl.ds(q * 16, 16)
                rows_v[e, sl] = rows_v[e, sl] * va0
            for q in range(4):
                sl = pl.ds(64 + q * 16, 16)
                rows_v[e, sl] = rows_v[e, sl] * va1
            return ecarry
        lax.fori_loop(0, B, _edge, 0, unroll=8)
        pltpu.make_async_copy(rows_v, agg_sh.at[ridxsc_v], ss[5]).start(add=True)

    def _wait_scatter(buf, ss):
        pltpu.make_async_copy(buf[12], agg_sh.at[buf[13]], ss[5]).wait()

    _fire_idx(0, bufs[0], sems[0])
    _fire_idx(1, bufs[1], sems[1])
    _fire_gathers(0, bufs[0], sems[0])

    def _group(g, carry):
        for b in range(3):
            k = g * 3 + b
            nb = (b + 1) % 3
            nb2 = (b + 2) % 3
            @pl.when(k >= 2)
            def _drain():
                _wait_scatter(bufs[nb], sems[nb])
            _fire_gathers(k + 1, bufs[nb], sems[nb])
            @pl.when(k < NCHUNK - 2)
            def _prefetch():
                _fire_idx(k + 2, bufs[nb2], sems[nb2])
            _process(bufs[b], sems[b])
        return carry
    lax.fori_loop(0, (NCHUNK - 1) // 3, _group, 0)
    _process(bufs[0], sems[0])
    for b in range(3):
        _wait_scatter(bufs[b], sems[b])
    plsc.subcore_barrier()

    def _silu_rows(nrows):
        def _silu_row(i, carry):
            for q in range(8):
                sl = pl.ds(q * 16, 16)
                v = wb_v[i, sl]
                wb_v[i, sl] = v / (1.0 + jnp.exp(-v))
            return carry
        lax.fori_loop(0, nrows, _silu_row, 0)

    for t in range(NWB):
        r0 = sid * RPT + t * RW
        pltpu.sync_copy(agg_sh.at[pl.ds(r0, RW)], wb_v)
        _silu_rows(RW)
        pltpu.sync_copy(wb_v, out_hbm.at[pl.ds(r0, RW), pl.ds(c * HALF, HALF)])

    @pl.when(sid == 0)
    def _wb_tail():
        pltpu.sync_copy(agg_sh.at[pl.ds(TAIL0, TAILN)], wb_v.at[pl.ds(0, TAILN)])
        _silu_rows(TAILN)
        pltpu.sync_copy(wb_v.at[pl.ds(0, TAILN)],
                        out_hbm.at[pl.ds(TAIL0, TAILN), pl.ds(c * HALF, HALF)])


_sc_call = pl.kernel(
    _sc_body,
    out_type=jax.ShapeDtypeStruct((N, OUT), jnp.float32),
    mesh=plsc.VectorSubcoreMesh(
        core_axis_name="c", subcore_axis_name="s",
        num_cores=NC, num_subcores=NS),
    scratch_types=[
        [
            [
                pltpu.VMEM((B,), jnp.int32),
                pltpu.VMEM((B,), jnp.int32),
                pltpu.VMEM((B,), jnp.int32),
                pltpu.VMEM((B,), jnp.int32),
                pltpu.VMEM((B,), jnp.int32),
                pltpu.VMEM((B,), jnp.int32),
                pltpu.VMEM((B,), jnp.float32),
                pltpu.VMEM((B,), jnp.float32),
                pltpu.VMEM((B,), jnp.float32),
                pltpu.VMEM((B,), jnp.float32),
                pltpu.VMEM((B + 16,), jnp.float32),
                pltpu.VMEM((B + 16,), jnp.float32),
                pltpu.VMEM((B, HALF), jnp.float32),
                pltpu.VMEM((B,), jnp.int32),
            ]
            for _ in range(3)
        ],
        pltpu.VMEM((RW, HALF), jnp.float32),
        pltpu.VMEM_SHARED((N, HALF), jnp.float32),
        [[pltpu.SemaphoreType.DMA] * 8 for _ in range(3)],
    ],
)


def kernel(x, edge_index, W_w, W_b, a_w, a_b, lin_w, lin_b):
    x = x.astype(jnp.float32)
    s32 = edge_index[0].astype(jnp.int32)
    r32 = edge_index[1].astype(jnp.int32)

    a_ws, a_wr = a_w[:, :QH], a_w[:, QH:]
    mats, biases = [], []
    for c in (0, 1):
        P = jnp.concatenate([a_ws[2 * c:2 * c + 2], a_wr[2 * c:2 * c + 2]], 0)
        mats.append(P @ W_w)
        bias = P @ W_b
        bias = bias.at[2].add(a_b[2 * c]).at[3].add(a_b[2 * c + 1])
        biases.append(bias)
    WBIG = jnp.concatenate([lin_w] + mats, 0)
    bbig = jnp.concatenate([lin_b] + biases, 0)[None, :]

    h_perm, anode = _tc_call(x, WBIG, bbig)
    return _sc_call(h_perm.reshape(2 * N, HALF),
                    anode.reshape(2 * N * ACOLS), s32, r32)


# speedup vs baseline: 1.9674x; 1.0161x over previous
import jax
import jax.numpy as jnp
from jax import lax
from jax.experimental import pallas as pl
from jax.experimental.pallas import tpu as pltpu
from jax.experimental.pallas import tpu_sc as plsc

N = 10000
D = 128
H = 4
Q = 8
QH = Q * H
OUT = 256
HALF = OUT // 2
E = 320000

NC = 2
NS = 16
EPT = E // NS
B = 80
NCHUNK = EPT // B

RPT = 624
RW = 48
NWB = RPT // RW
TAIL0 = NS * RPT
TAILN = N - TAIL0

BLK = 1000
ACOLS = 8
WCOLS = OUT + 8


def _tc_body(x_ref, lw_ref, lb_ref, ww_ref, wb_ref, aw_ref, ab_ref,
             h_ref, an_ref):
    x = x_ref[:]
    h = jnp.dot(x, lw_ref[:].T, preferred_element_type=jnp.float32)
    h = h + lb_ref[:]
    h_ref[0] = h[:, :HALF]
    h_ref[1] = h[:, HALF:OUT]
    attr = jnp.dot(x, ww_ref[:].T, preferred_element_type=jnp.float32)
    attr = attr + wb_ref[:]
    aw = aw_ref[:]
    ab = ab_ref[:]
    z = jnp.zeros((BLK, ACOLS - 4), jnp.float32)
    for cc in (0, 1):
        P = jnp.concatenate(
            [aw[2 * cc:2 * cc + 2, :QH], aw[2 * cc:2 * cc + 2, QH:]], axis=0)
        a4 = jnp.dot(attr, P.T, preferred_element_type=jnp.float32)
        bias = jnp.concatenate(
            [jnp.zeros((1, 2), jnp.float32), ab[:, 2 * cc:2 * cc + 2]], axis=1)
        an_ref[cc] = jnp.concatenate([a4 + bias, z], axis=1)


_tc_call = pl.pallas_call(
    _tc_body,
    grid=(N // BLK,),
    in_specs=[
        pl.BlockSpec((BLK, D), lambda j: (j, 0)),
        pl.BlockSpec((OUT, D), lambda j: (0, 0)),
        pl.BlockSpec((1, OUT), lambda j: (0, 0)),
        pl.BlockSpec((QH, D), lambda j: (0, 0)),
        pl.BlockSpec((1, QH), lambda j: (0, 0)),
        pl.BlockSpec((H, 2 * QH), lambda j: (0, 0)),
        pl.BlockSpec((1, H), lambda j: (0, 0)),
    ],
    out_specs=[
        pl.BlockSpec((2, BLK, HALF), lambda j: (0, j, 0)),
        pl.BlockSpec((2, BLK, ACOLS), lambda j: (0, j, 0)),
    ],
    out_shape=[
        jax.ShapeDtypeStruct((2, N, HALF), jnp.float32),
        jax.ShapeDtypeStruct((2, N, ACOLS), jnp.float32),
    ],
)


def _sc_body(hperm, anode, s_hbm, r_hbm, out_hbm,
             bufs, wb_v, agg_sh, sems):
    c = lax.axis_index("c")
    sid = lax.axis_index("s")
    cN = c * N

    def _zero_row(i, carry):
        for q in range(8):
            wb_v[i, pl.ds(q * 16, 16)] = jnp.zeros((16,), jnp.float32)
        return carry
    lax.fori_loop(0, RW, _zero_row, 0)
    for t in range(NWB):
        pltpu.sync_copy(wb_v, agg_sh.at[pl.ds(sid * RPT + t * RW, RW)])
    @pl.when(sid == 0)
    def _zero_tail():
        pltpu.sync_copy(wb_v.at[pl.ds(0, TAILN)], agg_sh.at[pl.ds(TAIL0, TAILN)])
    plsc.subcore_barrier()

    def _fire_idx(k, buf, ss):
        sidx_v, ridx_v = buf[0], buf[1]
        base = sid * EPT + k * B
        pltpu.async_copy(s_hbm.at[pl.ds(base, B)], sidx_v, ss[6])
        pltpu.async_copy(r_hbm.at[pl.ds(base, B)], ridx_v, ss[7])

    def _fire_gathers(k, buf, ss):
        (sidx_v, ridx_v, i0_v, i1_v, i2_v, i3_v,
         as0_v, as1_v, ar0_v, ar1_v, a0b_v, a1b_v, rows_v, ridxsc_v) = buf
        base = sid * EPT + k * B
        pltpu.make_async_copy(s_hbm.at[pl.ds(base, B)], sidx_v, ss[6]).wait()
        pltpu.make_async_copy(r_hbm.at[pl.ds(base, B)], ridx_v, ss[7]).wait()
        for t in range(B // 16):
            sl = pl.ds(t * 16, 16)
            sa = sidx_v[sl] + cN
            sidx_v[sl] = sa
            s8 = sa * ACOLS
            i0_v[sl] = s8
            i1_v[sl] = s8 + 1
            rv = ridx_v[sl]
            ridxsc_v[sl] = rv
            r8 = (rv + cN) * ACOLS
            i2_v[sl] = r8 + 2
            i3_v[sl] = r8 + 3
        pltpu.async_copy(hperm.at[sidx_v], rows_v, ss[0])
        pltpu.async_copy(anode.at[i0_v], as0_v, ss[1])
        pltpu.async_copy(anode.at[i1_v], as1_v, ss[2])
        pltpu.async_copy(anode.at[i2_v], ar0_v, ss[3])
        pltpu.async_copy(anode.at[i3_v], ar1_v, ss[4])

    def _process(buf, ss):
        (sidx_v, ridx_v, i0_v, i1_v, i2_v, i3_v,
         as0_v, as1_v, ar0_v, ar1_v, a0b_v, a1b_v, rows_v, ridxsc_v) = buf
        pltpu.make_async_copy(anode.at[i0_v], as0_v, ss[1]).wait()
        pltpu.make_async_copy(anode.at[i1_v], as1_v, ss[2]).wait()
        pltpu.make_async_copy(anode.at[i2_v], ar0_v, ss[3]).wait()
        pltpu.make_async_copy(anode.at[i3_v], ar1_v, ss[4]).wait()
        for t in range(B // 16):
            sl = pl.ds(t * 16, 16)
            a0b_v[sl] = as0_v[sl] + ar0_v[sl]
            a1b_v[sl] = as1_v[sl] + ar1_v[sl]
        pltpu.make_async_copy(hperm.at[sidx_v], rows_v, ss[0]).wait()

        def _edge(e, ecarry):
            va0 = jnp.full((16,), a0b_v[pl.ds(e, 16)][0], jnp.float32)
            va1 = jnp.full((16,), a1b_v[pl.ds(e, 16)][0], jnp.float32)
            for q in range(4):
                sl = pl.ds(q * 16, 16)
                rows_v[e, sl] = rows_v[e, sl] * va0
            for q in range(4):
                sl = pl.ds(64 + q * 16, 16)
                rows_v[e, sl] = rows_v[e, sl] * va1
            return ecarry
        lax.fori_loop(0, B, _edge, 0, unroll=8)
        pltpu.make_async_copy(rows_v, agg_sh.at[ridxsc_v], ss[5]).start(add=True)

    def _wait_scatter(buf, ss):
        pltpu.make_async_copy(buf[12], agg_sh.at[buf[13]], ss[5]).wait()

    _fire_idx(0, bufs[0], sems[0])
    _fire_idx(1, bufs[1], sems[1])
    _fire_gathers(0, bufs[0], sems[0])

    def _group(g, carry):
        for b in range(3):
            k = g * 3 + b
            nb = (b + 1) % 3
            nb2 = (b + 2) % 3
            @pl.when(k >= 2)
            def _drain():
                _wait_scatter(bufs[nb], sems[nb])
            _fire_gathers(k + 1, bufs[nb], sems[nb])
            @pl.when(k < NCHUNK - 2)
            def _prefetch():
                _fire_idx(k + 2, bufs[nb2], sems[nb2])
            _process(bufs[b], sems[b])
        return carry
    lax.fori_loop(0, (NCHUNK - 1) // 3, _group, 0)
    _process(bufs[0], sems[0])
    for b in range(3):
        _wait_scatter(bufs[b], sems[b])
    plsc.subcore_barrier()

    def _silu_rows(nrows):
        def _silu_row(i, carry):
            for q in range(8):
                sl = pl.ds(q * 16, 16)
                v = wb_v[i, sl]
                wb_v[i, sl] = v / (1.0 + jnp.exp(-v))
            return carry
        lax.fori_loop(0, nrows, _silu_row, 0)

    for t in range(NWB):
        r0 = sid * RPT + t * RW
        pltpu.sync_copy(agg_sh.at[pl.ds(r0, RW)], wb_v)
        _silu_rows(RW)
        pltpu.sync_copy(wb_v, out_hbm.at[pl.ds(r0, RW), pl.ds(c * HALF, HALF)])

    @pl.when(sid == 0)
    def _wb_tail():
        pltpu.sync_copy(agg_sh.at[pl.ds(TAIL0, TAILN)], wb_v.at[pl.ds(0, TAILN)])
        _silu_rows(TAILN)
        pltpu.sync_copy(wb_v.at[pl.ds(0, TAILN)],
                        out_hbm.at[pl.ds(TAIL0, TAILN), pl.ds(c * HALF, HALF)])


_sc_call = pl.kernel(
    _sc_body,
    out_type=jax.ShapeDtypeStruct((N, OUT), jnp.float32),
    mesh=plsc.VectorSubcoreMesh(
        core_axis_name="c", subcore_axis_name="s",
        num_cores=NC, num_subcores=NS),
    scratch_types=[
        [
            [
                pltpu.VMEM((B,), jnp.int32),
                pltpu.VMEM((B,), jnp.int32),
                pltpu.VMEM((B,), jnp.int32),
                pltpu.VMEM((B,), jnp.int32),
                pltpu.VMEM((B,), jnp.int32),
                pltpu.VMEM((B,), jnp.int32),
                pltpu.VMEM((B,), jnp.float32),
                pltpu.VMEM((B,), jnp.float32),
                pltpu.VMEM((B,), jnp.float32),
                pltpu.VMEM((B,), jnp.float32),
                pltpu.VMEM((B + 16,), jnp.float32),
                pltpu.VMEM((B + 16,), jnp.float32),
                pltpu.VMEM((B, HALF), jnp.float32),
                pltpu.VMEM((B,), jnp.int32),
            ]
            for _ in range(3)
        ],
        pltpu.VMEM((RW, HALF), jnp.float32),
        pltpu.VMEM_SHARED((N, HALF), jnp.float32),
        [[pltpu.SemaphoreType.DMA] * 8 for _ in range(3)],
    ],
)


def kernel(x, edge_index, W_w, W_b, a_w, a_b, lin_w, lin_b):
    x = x.astype(jnp.float32)
    s32 = edge_index[0].astype(jnp.int32)
    r32 = edge_index[1].astype(jnp.int32)

    h_perm, anode = _tc_call(
        x, lin_w, lin_b.reshape(1, OUT), W_w, W_b.reshape(1, QH),
        a_w, a_b.reshape(1, H))
    return _sc_call(h_perm.reshape(2 * N, HALF),
                    anode.reshape(2 * N * ACOLS), s32, r32)
